# baseline (speedup 1.0000x reference)
import functools

import jax
import jax.numpy as jnp
import numpy as np
from jax.experimental import pallas as pl

_NUM_CLASSES = 80
_ANCHORS = np.array(
    [[10, 13], [16, 30], [33, 23], [30, 61], [62, 45], [59, 119],
     [116, 90], [156, 198], [373, 326]], dtype=np.float32)
_MASK = [[0, 1, 2], [3, 4, 5], [6, 7, 8]]
_STRIDES = [8.0, 16.0, 32.0]
_MAX_OUT = 100
_IOU_THR = 0.5
_SCORE_THR = 0.25
_PRESELECT = 400
_PAD = 512

_LEVELS = [(64, 64), (32, 32), (16, 16)]
_N_TOTAL = sum(h * w * 3 for h, w in _LEVELS)


def _build_consts():
    gx, gy, aw, ah, st = [], [], [], [], []
    for i, (H, W) in enumerate(_LEVELS):
        anc = _ANCHORS[_MASK[i]]
        hh, ww, aa = np.meshgrid(np.arange(H), np.arange(W), np.arange(3),
                                 indexing="ij")
        gx.append(ww.reshape(-1).astype(np.float32))
        gy.append(hh.reshape(-1).astype(np.float32))
        aw.append(anc[aa.reshape(-1), 0])
        ah.append(anc[aa.reshape(-1), 1])
        st.append(np.full(H * W * 3, _STRIDES[i], dtype=np.float32))
    rows = [np.concatenate(v) for v in (gx, gy, aw, ah, st)]
    rows += [np.zeros(_N_TOTAL, np.float32)] * 3
    return np.stack(rows, axis=0)


_CONSTS = _build_consts()


_BITS_LO = 0x3E800000
_BITS_HI = 0x3F800000


def _decode_kernel(x_ref, c_ref, b_ref, s_ref, t_ref):
    x = x_ref[0]
    c = c_ref[...]
    xy = jax.nn.sigmoid(x[0:2, :])
    ctr = (xy + c[0:2, :]) * c[4:5, :]
    half = jnp.exp(x[2:4, :]) * c[2:4, :] * 0.5
    b_ref[0, 0:2, :] = ctr - half
    b_ref[0, 2:4, :] = ctr + half
    obj = jax.nn.sigmoid(x[4:5, :])
    sc = obj * jax.nn.sigmoid(x[5:85, :])
    s_ref[0] = sc

    ms = jnp.where(sc > _SCORE_THR, sc, -1.0)
    lo0 = jnp.full((_NUM_CLASSES, 1), _BITS_LO, jnp.int32)
    hi0 = jnp.full((_NUM_CLASSES, 1), _BITS_HI, jnp.int32)

    def bbody(_, lh):
        lo, hi = lh
        mid = (lo + hi) >> 1
        mid_f = jax.lax.bitcast_convert_type(mid, jnp.float32)
        cnt = jnp.sum((ms >= mid_f).astype(jnp.int32), axis=1, keepdims=True)
        ge = cnt >= _PRESELECT
        return jnp.where(ge, mid, lo), jnp.where(ge, hi, mid)

    lo, _ = jax.lax.fori_loop(0, 24, bbody, (lo0, hi0))
    t = jax.lax.bitcast_convert_type(lo, jnp.float32)
    t_ref[0] = jnp.broadcast_to(t, (_NUM_CLASSES, 128))


def _nms_kernel(s_ref, x1_ref, y1_ref, x2_ref, y2_ref,
                os_ref, ox1_ref, oy1_ref, ox2_ref, oy2_ref):
    s = s_ref[0]
    cur0 = jnp.where(s > _SCORE_THR, s, -1.0)
    x1 = x1_ref[0]
    y1 = y1_ref[0]
    x2 = x2_ref[0]
    y2 = y2_ref[0]
    a2 = (x2 - x1) * (y2 - y1)
    iota = jax.lax.broadcasted_iota(jnp.int32, (_NUM_CLASSES, _PAD), 1)
    io_m = jax.lax.broadcasted_iota(jnp.int32, (_NUM_CLASSES, 128), 1)
    zm = jnp.zeros((_NUM_CLASSES, 128), jnp.float32)

    def body(i, carry):
        cur, os_, o1, o2, o3, o4 = carry
        m = jnp.max(cur, axis=1, keepdims=True)
        sel_first = jnp.min(jnp.where(cur == m, iota, _PAD), axis=1,
                            keepdims=True)
        sel = iota == sel_first
        bx1 = jnp.sum(jnp.where(sel, x1, 0.0), axis=1, keepdims=True)
        by1 = jnp.sum(jnp.where(sel, y1, 0.0), axis=1, keepdims=True)
        bx2 = jnp.sum(jnp.where(sel, x2, 0.0), axis=1, keepdims=True)
        by2 = jnp.sum(jnp.where(sel, y2, 0.0), axis=1, keepdims=True)
        ok = m > 0.0
        wr = io_m == i
        os_ = jnp.where(wr, jnp.where(ok, m, 0.0), os_)
        o1 = jnp.where(wr, jnp.where(ok, bx1, 0.0), o1)
        o2 = jnp.where(wr, jnp.where(ok, by1, 0.0), o2)
        o3 = jnp.where(wr, jnp.where(ok, bx2, 0.0), o3)
        o4 = jnp.where(wr, jnp.where(ok, by2, 0.0), o4)
        iw = jnp.maximum(jnp.minimum(bx2, x2) - jnp.maximum(bx1, x1), 0.0)
        ih = jnp.maximum(jnp.minimum(by2, y2) - jnp.maximum(by1, y1), 0.0)
        inter = iw * ih
        a1 = (bx2 - bx1) * (by2 - by1)
        iou = inter / (a1 + a2 - inter + 1e-9)
        cur = jnp.where(iou > _IOU_THR, -1.0, cur)
        cur = jnp.where(sel, -1.0, cur)
        return (cur, os_, o1, o2, o3, o4)

    cur, os_, o1, o2, o3, o4 = jax.lax.fori_loop(
        0, _MAX_OUT, body, (cur0, zm, zm, zm, zm, zm))
    os_ref[0] = os_
    ox1_ref[0] = o1
    oy1_ref[0] = o2
    ox2_ref[0] = o3
    oy2_ref[0] = o4


@jax.jit
def kernel(logits_0, logits_1, logits_2):
    B = logits_0.shape[0]
    N = _N_TOTAL
    parts = []
    for lg, (H, W) in zip((logits_0, logits_1, logits_2), _LEVELS):
        parts.append(lg.reshape(B, H * W * 3, 85))
    X = jnp.concatenate(parts, axis=1).transpose(0, 2, 1)
    consts = jnp.asarray(_CONSTS)

    boxes_t, scores_t, t_out = pl.pallas_call(
        _decode_kernel,
        grid=(B,),
        in_specs=[
            pl.BlockSpec((1, 85, N), lambda b: (b, 0, 0)),
            pl.BlockSpec((8, N), lambda b: (0, 0)),
        ],
        out_specs=[
            pl.BlockSpec((1, 4, N), lambda b: (b, 0, 0)),
            pl.BlockSpec((1, _NUM_CLASSES, N), lambda b: (b, 0, 0)),
            pl.BlockSpec((1, _NUM_CLASSES, 128), lambda b: (b, 0, 0)),
        ],
        out_shape=[
            jax.ShapeDtypeStruct((B, 4, N), jnp.float32),
            jax.ShapeDtypeStruct((B, _NUM_CLASSES, N), jnp.float32),
            jax.ShapeDtypeStruct((B, _NUM_CLASSES, 128), jnp.float32),
        ],
    )(X, consts)

    R = B * _NUM_CLASSES
    NCH = N // 128
    sflat = scores_t.reshape(R, N)
    t = t_out[:, :, 0].reshape(R, 1)
    mask = (sflat > _SCORE_THR) & (sflat >= t)
    mi = mask.astype(jnp.int32)
    ccum = jnp.cumsum(mi.reshape(R, NCH, 128).sum(-1), axis=1)
    total = ccum[:, -1:]
    k = jnp.arange(_PRESELECT, dtype=jnp.int32)
    chunk_idx = (ccum[:, :, None] <= k[None, None, :]).astype(jnp.int32).sum(1)
    ci = jnp.minimum(chunk_idx, NCH - 1)
    bprev = jnp.take_along_axis(ccum, jnp.clip(chunk_idx - 1, 0, NCH - 1),
                                axis=1)
    base = jnp.where(chunk_idx > 0, bprev, 0)
    j = k[None, :] - base
    selc = jnp.take_along_axis(mi.reshape(R, NCH, 128), ci[:, :, None],
                               axis=1)
    lane_cum = jnp.cumsum(selc, axis=2)
    lane_idx = (lane_cum <= j[:, :, None]).astype(jnp.int32).sum(2)
    validk = k[None, :] < total
    top_i = jnp.where(validk,
                      ci * 128 + jnp.minimum(lane_idx, 127), 0)
    top_s = jnp.where(validk, jnp.take_along_axis(sflat, top_i, axis=1), 0.0)
    bi = top_i.reshape(B, _NUM_CLASSES, _PRESELECT)
    bb = boxes_t[jnp.arange(B)[:, None, None, None],
                 jnp.arange(4)[None, None, :, None],
                 bi[:, :, None, :]]
    s_pad = jnp.zeros((B, _NUM_CLASSES, _PAD), jnp.float32)
    s_pad = s_pad.at[:, :, :_PRESELECT].set(
        top_s.reshape(B, _NUM_CLASSES, _PRESELECT))
    b_pad = jnp.zeros((B, _NUM_CLASSES, 4, _PAD), jnp.float32)
    b_pad = b_pad.at[:, :, :, :_PRESELECT].set(bb)
    x1p = b_pad[:, :, 0, :]
    y1p = b_pad[:, :, 1, :]
    x2p = b_pad[:, :, 2, :]
    y2p = b_pad[:, :, 3, :]

    spec_in = pl.BlockSpec((1, _NUM_CLASSES, _PAD), lambda b: (b, 0, 0))
    spec_out = pl.BlockSpec((1, _NUM_CLASSES, 128), lambda b: (b, 0, 0))
    outs = pl.pallas_call(
        _nms_kernel,
        grid=(B,),
        in_specs=[spec_in] * 5,
        out_specs=[spec_out] * 5,
        out_shape=[jax.ShapeDtypeStruct((B, _NUM_CLASSES, 128), jnp.float32)
                   ] * 5,
    )(s_pad, x1p, y1p, x2p, y2p)
    ss, ox1, oy1, ox2, oy2 = outs

    flat_s = ss[:, :, :_MAX_OUT].reshape(B, _NUM_CLASSES * _MAX_OUT)
    flat_b = jnp.stack([ox1, oy1, ox2, oy2], axis=-1)[:, :, :_MAX_OUT, :]
    flat_b = flat_b.reshape(B, _NUM_CLASSES * _MAX_OUT, 4)
    top_s2, top_i2 = jax.lax.top_k(flat_s, _MAX_OUT)
    top_b = jnp.take_along_axis(flat_b, top_i2[:, :, None], axis=1)
    top_c = (top_i2 // _MAX_OUT).astype(jnp.float32)
    ok = top_s2 > 0.0
    top_b = jnp.where(ok[:, :, None], top_b, 0.0)
    top_c = jnp.where(ok, top_c, 0.0)
    top_s2 = jnp.where(ok, top_s2, 0.0)
    valid = jnp.sum(ok, axis=1).astype(jnp.int32)
    return top_b, top_s2, top_c, valid


# speedup vs baseline: 4.0024x; 4.0024x over previous
import functools

import jax
import jax.numpy as jnp
import numpy as np
from jax.experimental import pallas as pl

_NUM_CLASSES = 80
_ANCHORS = np.array(
    [[10, 13], [16, 30], [33, 23], [30, 61], [62, 45], [59, 119],
     [116, 90], [156, 198], [373, 326]], dtype=np.float32)
_MASK = [[0, 1, 2], [3, 4, 5], [6, 7, 8]]
_STRIDES = [8.0, 16.0, 32.0]
_MAX_OUT = 100
_IOU_THR = 0.5
_SCORE_THR = 0.25
_PRESELECT = 400
_PAD = 512

_LEVELS = [(64, 64), (32, 32), (16, 16)]
_N_TOTAL = sum(h * w * 3 for h, w in _LEVELS)


def _build_consts():
    gx, gy, aw, ah, st = [], [], [], [], []
    for i, (H, W) in enumerate(_LEVELS):
        anc = _ANCHORS[_MASK[i]]
        hh, ww, aa = np.meshgrid(np.arange(H), np.arange(W), np.arange(3),
                                 indexing="ij")
        gx.append(ww.reshape(-1).astype(np.float32))
        gy.append(hh.reshape(-1).astype(np.float32))
        aw.append(anc[aa.reshape(-1), 0])
        ah.append(anc[aa.reshape(-1), 1])
        st.append(np.full(H * W * 3, _STRIDES[i], dtype=np.float32))
    rows = [np.concatenate(v) for v in (gx, gy, aw, ah, st)]
    rows += [np.zeros(_N_TOTAL, np.float32)] * 3
    return np.stack(rows, axis=0)


_CONSTS = _build_consts()


_BITS_LO = 0x3E800000
_BITS_HI = 0x3F800000


def _decode_kernel(x_ref, c_ref, b_ref, s_ref, t_ref):
    x = x_ref[0]
    c = c_ref[...]
    xy = jax.nn.sigmoid(x[0:2, :])
    ctr = (xy + c[0:2, :]) * c[4:5, :]
    half = jnp.exp(x[2:4, :]) * c[2:4, :] * 0.5
    b_ref[0, 0:2, :] = ctr - half
    b_ref[0, 2:4, :] = ctr + half
    obj = jax.nn.sigmoid(x[4:5, :])
    sc = obj * jax.nn.sigmoid(x[5:85, :])
    s_ref[0] = sc

    ms = jnp.where(sc > _SCORE_THR, sc, -1.0)
    lo0 = jnp.full((_NUM_CLASSES, 1), _BITS_LO, jnp.int32)
    hi0 = jnp.full((_NUM_CLASSES, 1), _BITS_HI, jnp.int32)

    def bbody(_, lh):
        lo, hi = lh
        mid = (lo + hi) >> 1
        mid_f = jax.lax.bitcast_convert_type(mid, jnp.float32)
        cnt = jnp.sum((ms >= mid_f).astype(jnp.int32), axis=1, keepdims=True)
        ge = cnt >= _PRESELECT
        return jnp.where(ge, mid, lo), jnp.where(ge, hi, mid)

    lo, _ = jax.lax.fori_loop(0, 24, bbody, (lo0, hi0))
    t = jax.lax.bitcast_convert_type(lo, jnp.float32)
    t_ref[0] = jnp.broadcast_to(t, (_NUM_CLASSES, 128))


def _nms_kernel(s_ref, x1_ref, y1_ref, x2_ref, y2_ref,
                os_ref, ox1_ref, oy1_ref, ox2_ref, oy2_ref):
    s = s_ref[0]
    cur0 = jnp.where(s > _SCORE_THR, s, -1.0)
    x1 = x1_ref[0]
    y1 = y1_ref[0]
    x2 = x2_ref[0]
    y2 = y2_ref[0]
    a2 = (x2 - x1) * (y2 - y1)
    iota = jax.lax.broadcasted_iota(jnp.int32, (_NUM_CLASSES, _PAD), 1)
    io_m = jax.lax.broadcasted_iota(jnp.int32, (_NUM_CLASSES, 128), 1)
    zm = jnp.zeros((_NUM_CLASSES, 128), jnp.float32)

    def body(i, carry):
        cur, os_, o1, o2, o3, o4 = carry
        m = jnp.max(cur, axis=1, keepdims=True)
        sel_first = jnp.min(jnp.where(cur == m, iota, _PAD), axis=1,
                            keepdims=True)
        sel = iota == sel_first
        bx1 = jnp.sum(jnp.where(sel, x1, 0.0), axis=1, keepdims=True)
        by1 = jnp.sum(jnp.where(sel, y1, 0.0), axis=1, keepdims=True)
        bx2 = jnp.sum(jnp.where(sel, x2, 0.0), axis=1, keepdims=True)
        by2 = jnp.sum(jnp.where(sel, y2, 0.0), axis=1, keepdims=True)
        ok = m > 0.0
        wr = io_m == i
        os_ = jnp.where(wr, jnp.where(ok, m, 0.0), os_)
        o1 = jnp.where(wr, jnp.where(ok, bx1, 0.0), o1)
        o2 = jnp.where(wr, jnp.where(ok, by1, 0.0), o2)
        o3 = jnp.where(wr, jnp.where(ok, bx2, 0.0), o3)
        o4 = jnp.where(wr, jnp.where(ok, by2, 0.0), o4)
        iw = jnp.maximum(jnp.minimum(bx2, x2) - jnp.maximum(bx1, x1), 0.0)
        ih = jnp.maximum(jnp.minimum(by2, y2) - jnp.maximum(by1, y1), 0.0)
        inter = iw * ih
        a1 = (bx2 - bx1) * (by2 - by1)
        iou = inter / (a1 + a2 - inter + 1e-9)
        cur = jnp.where(iou > _IOU_THR, -1.0, cur)
        cur = jnp.where(sel, -1.0, cur)
        return (cur, os_, o1, o2, o3, o4)

    cur, os_, o1, o2, o3, o4 = jax.lax.fori_loop(
        0, _MAX_OUT, body, (cur0, zm, zm, zm, zm, zm))
    os_ref[0] = os_
    ox1_ref[0] = o1
    oy1_ref[0] = o2
    ox2_ref[0] = o3
    oy2_ref[0] = o4


@jax.jit
def kernel(logits_0, logits_1, logits_2):
    B = logits_0.shape[0]
    N = _N_TOTAL
    parts = []
    for lg, (H, W) in zip((logits_0, logits_1, logits_2), _LEVELS):
        parts.append(lg.reshape(B, H * W * 3, 85))
    X = jnp.concatenate(parts, axis=1).transpose(0, 2, 1)
    consts = jnp.asarray(_CONSTS)

    boxes_t, scores_t, t_out = pl.pallas_call(
        _decode_kernel,
        grid=(B,),
        in_specs=[
            pl.BlockSpec((1, 85, N), lambda b: (b, 0, 0)),
            pl.BlockSpec((8, N), lambda b: (0, 0)),
        ],
        out_specs=[
            pl.BlockSpec((1, 4, N), lambda b: (b, 0, 0)),
            pl.BlockSpec((1, _NUM_CLASSES, N), lambda b: (b, 0, 0)),
            pl.BlockSpec((1, _NUM_CLASSES, 128), lambda b: (b, 0, 0)),
        ],
        out_shape=[
            jax.ShapeDtypeStruct((B, 4, N), jnp.float32),
            jax.ShapeDtypeStruct((B, _NUM_CLASSES, N), jnp.float32),
            jax.ShapeDtypeStruct((B, _NUM_CLASSES, 128), jnp.float32),
        ],
    )(X, consts)

    R = B * _NUM_CLASSES
    NCH = N // 128
    sflat = scores_t.reshape(R, N)
    t = t_out[:, :, 0].reshape(R, 1)
    mask = (sflat > _SCORE_THR) & (sflat >= t)
    mf = mask.astype(jnp.float32)
    mch = mf.reshape(R, NCH, 128)
    cnt = mch.sum(-1)
    ccum = jnp.cumsum(cnt, axis=1)
    total = ccum[:, -1:]
    kf = jnp.arange(_PRESELECT, dtype=jnp.float32)
    cmp = (ccum[:, :, None] <= kf[None, None, :]).astype(jnp.float32)
    chunk_idx = cmp.sum(1)
    base = (cnt[:, :, None] * cmp).sum(1)
    ci = jnp.minimum(chunk_idx, float(NCH - 1))
    onehot = (jnp.arange(NCH, dtype=jnp.float32)[None, None, :]
              == ci[:, :, None]).astype(jnp.float32)
    selc = jnp.einsum('rkc,rcl->rkl', onehot, mch)
    lane_cum = jnp.cumsum(selc, axis=2)
    j = kf[None, :] - base
    lane_idx = (lane_cum <= j[:, :, None]).astype(jnp.float32).sum(2)
    validk = kf[None, :] < total
    idx_f = ci * 128.0 + jnp.minimum(lane_idx, 127.0)
    top_i = jnp.where(validk, idx_f, 0.0).astype(jnp.int32)
    top_s = jnp.where(validk, jnp.take_along_axis(sflat, top_i, axis=1), 0.0)
    bi = top_i.reshape(B, _NUM_CLASSES, _PRESELECT)
    bb = boxes_t[jnp.arange(B)[:, None, None, None],
                 jnp.arange(4)[None, None, :, None],
                 bi[:, :, None, :]]
    s_pad = jnp.zeros((B, _NUM_CLASSES, _PAD), jnp.float32)
    s_pad = s_pad.at[:, :, :_PRESELECT].set(
        top_s.reshape(B, _NUM_CLASSES, _PRESELECT))
    b_pad = jnp.zeros((B, _NUM_CLASSES, 4, _PAD), jnp.float32)
    b_pad = b_pad.at[:, :, :, :_PRESELECT].set(bb)
    x1p = b_pad[:, :, 0, :]
    y1p = b_pad[:, :, 1, :]
    x2p = b_pad[:, :, 2, :]
    y2p = b_pad[:, :, 3, :]

    spec_in = pl.BlockSpec((1, _NUM_CLASSES, _PAD), lambda b: (b, 0, 0))
    spec_out = pl.BlockSpec((1, _NUM_CLASSES, 128), lambda b: (b, 0, 0))
    outs = pl.pallas_call(
        _nms_kernel,
        grid=(B,),
        in_specs=[spec_in] * 5,
        out_specs=[spec_out] * 5,
        out_shape=[jax.ShapeDtypeStruct((B, _NUM_CLASSES, 128), jnp.float32)
                   ] * 5,
    )(s_pad, x1p, y1p, x2p, y2p)
    ss, ox1, oy1, ox2, oy2 = outs

    flat_s = ss[:, :, :_MAX_OUT].reshape(B, _NUM_CLASSES * _MAX_OUT)
    flat_b = jnp.stack([ox1, oy1, ox2, oy2], axis=-1)[:, :, :_MAX_OUT, :]
    flat_b = flat_b.reshape(B, _NUM_CLASSES * _MAX_OUT, 4)
    top_s2, top_i2 = jax.lax.top_k(flat_s, _MAX_OUT)
    top_b = jnp.take_along_axis(flat_b, top_i2[:, :, None], axis=1)
    top_c = (top_i2 // _MAX_OUT).astype(jnp.float32)
    ok = top_s2 > 0.0
    top_b = jnp.where(ok[:, :, None], top_b, 0.0)
    top_c = jnp.where(ok, top_c, 0.0)
    top_s2 = jnp.where(ok, top_s2, 0.0)
    valid = jnp.sum(ok, axis=1).astype(jnp.int32)
    return top_b, top_s2, top_c, valid


# speedup vs baseline: 6.4542x; 1.6126x over previous
import functools

import jax
import jax.numpy as jnp
import numpy as np
from jax.experimental import pallas as pl

_NUM_CLASSES = 80
_ANCHORS = np.array(
    [[10, 13], [16, 30], [33, 23], [30, 61], [62, 45], [59, 119],
     [116, 90], [156, 198], [373, 326]], dtype=np.float32)
_MASK = [[0, 1, 2], [3, 4, 5], [6, 7, 8]]
_STRIDES = [8.0, 16.0, 32.0]
_MAX_OUT = 100
_IOU_THR = 0.5
_SCORE_THR = 0.25
_PRESELECT = 400
_PAD = 512

_LEVELS = [(64, 64), (32, 32), (16, 16)]
_N_TOTAL = sum(h * w * 3 for h, w in _LEVELS)


def _build_consts():
    gx, gy, aw, ah, st = [], [], [], [], []
    for i, (H, W) in enumerate(_LEVELS):
        anc = _ANCHORS[_MASK[i]]
        hh, ww, aa = np.meshgrid(np.arange(H), np.arange(W), np.arange(3),
                                 indexing="ij")
        gx.append(ww.reshape(-1).astype(np.float32))
        gy.append(hh.reshape(-1).astype(np.float32))
        aw.append(anc[aa.reshape(-1), 0])
        ah.append(anc[aa.reshape(-1), 1])
        st.append(np.full(H * W * 3, _STRIDES[i], dtype=np.float32))
    rows = [np.concatenate(v) for v in (gx, gy, aw, ah, st)]
    rows += [np.zeros(_N_TOTAL, np.float32)] * 3
    return np.stack(rows, axis=0)


_CONSTS = _build_consts()


_BITS_LO = 0x3E800000
_BITS_HI = 0x3F800000


def _decode_kernel(x_ref, c_ref, b_ref, s_ref, t_ref):
    x = x_ref[0]
    c = c_ref[...]
    xy = jax.nn.sigmoid(x[0:2, :])
    ctr = (xy + c[0:2, :]) * c[4:5, :]
    half = jnp.exp(x[2:4, :]) * c[2:4, :] * 0.5
    b_ref[0, 0:2, :] = ctr - half
    b_ref[0, 2:4, :] = ctr + half
    obj = jax.nn.sigmoid(x[4:5, :])
    sc = obj * jax.nn.sigmoid(x[5:85, :])
    s_ref[0] = sc

    ms = jnp.where(sc > _SCORE_THR, sc, -1.0)
    lo0 = jnp.full((_NUM_CLASSES, 1), _BITS_LO, jnp.int32)
    hi0 = jnp.full((_NUM_CLASSES, 1), _BITS_HI, jnp.int32)

    def bbody(_, lh):
        lo, hi = lh
        mid = (lo + hi) >> 1
        mid_f = jax.lax.bitcast_convert_type(mid, jnp.float32)
        cnt = jnp.sum((ms >= mid_f).astype(jnp.int32), axis=1, keepdims=True)
        ge = cnt >= _PRESELECT
        return jnp.where(ge, mid, lo), jnp.where(ge, hi, mid)

    lo, _ = jax.lax.fori_loop(0, 24, bbody, (lo0, hi0))
    t = jax.lax.bitcast_convert_type(lo, jnp.float32)
    t_ref[0] = jnp.broadcast_to(t, (_NUM_CLASSES, 128))


def _nms_kernel(s_ref, x1_ref, y1_ref, x2_ref, y2_ref,
                os_ref, ox1_ref, oy1_ref, ox2_ref, oy2_ref):
    s = s_ref[0]
    cur0 = jnp.where(s > _SCORE_THR, s, -1.0)
    x1 = x1_ref[0]
    y1 = y1_ref[0]
    x2 = x2_ref[0]
    y2 = y2_ref[0]
    a2 = (x2 - x1) * (y2 - y1)
    iota = jax.lax.broadcasted_iota(jnp.int32, (_NUM_CLASSES, _PAD), 1)
    io_m = jax.lax.broadcasted_iota(jnp.int32, (_NUM_CLASSES, 128), 1)
    zm = jnp.zeros((_NUM_CLASSES, 128), jnp.float32)

    def body(i, carry):
        cur, os_, o1, o2, o3, o4 = carry
        m = jnp.max(cur, axis=1, keepdims=True)
        sel_first = jnp.min(jnp.where(cur == m, iota, _PAD), axis=1,
                            keepdims=True)
        sel = iota == sel_first
        bx1 = jnp.sum(jnp.where(sel, x1, 0.0), axis=1, keepdims=True)
        by1 = jnp.sum(jnp.where(sel, y1, 0.0), axis=1, keepdims=True)
        bx2 = jnp.sum(jnp.where(sel, x2, 0.0), axis=1, keepdims=True)
        by2 = jnp.sum(jnp.where(sel, y2, 0.0), axis=1, keepdims=True)
        ok = m > 0.0
        wr = io_m == i
        os_ = jnp.where(wr, jnp.where(ok, m, 0.0), os_)
        o1 = jnp.where(wr, jnp.where(ok, bx1, 0.0), o1)
        o2 = jnp.where(wr, jnp.where(ok, by1, 0.0), o2)
        o3 = jnp.where(wr, jnp.where(ok, bx2, 0.0), o3)
        o4 = jnp.where(wr, jnp.where(ok, by2, 0.0), o4)
        iw = jnp.maximum(jnp.minimum(bx2, x2) - jnp.maximum(bx1, x1), 0.0)
        ih = jnp.maximum(jnp.minimum(by2, y2) - jnp.maximum(by1, y1), 0.0)
        inter = iw * ih
        a1 = (bx2 - bx1) * (by2 - by1)
        iou = inter / (a1 + a2 - inter + 1e-9)
        cur = jnp.where(iou > _IOU_THR, -1.0, cur)
        cur = jnp.where(sel, -1.0, cur)
        return (cur, os_, o1, o2, o3, o4)

    cur, os_, o1, o2, o3, o4 = jax.lax.fori_loop(
        0, _MAX_OUT, body, (cur0, zm, zm, zm, zm, zm))
    os_ref[0] = os_
    ox1_ref[0] = o1
    oy1_ref[0] = o2
    ox2_ref[0] = o3
    oy2_ref[0] = o4


@jax.jit
def kernel(logits_0, logits_1, logits_2):
    B = logits_0.shape[0]
    N = _N_TOTAL
    parts = []
    for lg, (H, W) in zip((logits_0, logits_1, logits_2), _LEVELS):
        parts.append(lg.reshape(B, H * W * 3, 85))
    X = jnp.concatenate(parts, axis=1).transpose(0, 2, 1)
    consts = jnp.asarray(_CONSTS)

    boxes_t, scores_t, t_out = pl.pallas_call(
        _decode_kernel,
        grid=(B,),
        in_specs=[
            pl.BlockSpec((1, 85, N), lambda b: (b, 0, 0)),
            pl.BlockSpec((8, N), lambda b: (0, 0)),
        ],
        out_specs=[
            pl.BlockSpec((1, 4, N), lambda b: (b, 0, 0)),
            pl.BlockSpec((1, _NUM_CLASSES, N), lambda b: (b, 0, 0)),
            pl.BlockSpec((1, _NUM_CLASSES, 128), lambda b: (b, 0, 0)),
        ],
        out_shape=[
            jax.ShapeDtypeStruct((B, 4, N), jnp.float32),
            jax.ShapeDtypeStruct((B, _NUM_CLASSES, N), jnp.float32),
            jax.ShapeDtypeStruct((B, _NUM_CLASSES, 128), jnp.float32),
        ],
    )(X, consts)

    R = B * _NUM_CLASSES
    NCH = N // 128
    sflat = scores_t.reshape(R, N)
    t = t_out[:, :, 0].reshape(R, 1)
    mask = (sflat > _SCORE_THR) & (sflat >= t)
    mch = mask.reshape(R, NCH, 128).astype(jnp.bfloat16)
    cnt = jnp.sum(mch, axis=-1, dtype=jnp.float32)
    ccum = jnp.cumsum(cnt, axis=1)
    total = ccum[:, -1:]
    kf = jnp.arange(_PRESELECT, dtype=jnp.float32)
    cmp = (ccum[:, :, None] <= kf[None, None, :]).astype(jnp.bfloat16)
    chunk_idx = jnp.sum(cmp, axis=1, dtype=jnp.float32)
    base = jnp.sum(cnt.astype(jnp.bfloat16)[:, :, None] * cmp, axis=1,
                   dtype=jnp.float32)
    ci = jnp.minimum(chunk_idx, float(NCH - 1))
    onehot = (jnp.arange(NCH, dtype=jnp.float32)[None, None, :]
              == ci[:, :, None]).astype(jnp.bfloat16)
    selc = jnp.einsum('rkc,rcl->rkl', onehot, mch,
                      preferred_element_type=jnp.bfloat16)
    lane_cum = jnp.cumsum(selc, axis=2)
    j = (kf[None, :] - base).astype(jnp.bfloat16)
    lane_idx = jnp.sum((lane_cum <= j[:, :, None]).astype(jnp.bfloat16),
                       axis=2, dtype=jnp.float32)
    validk = kf[None, :] < total
    idx_f = ci * 128.0 + jnp.minimum(lane_idx, 127.0)
    top_i = jnp.where(validk, idx_f, 0.0).astype(jnp.int32)
    top_s = jnp.where(validk, jnp.take_along_axis(sflat, top_i, axis=1), 0.0)
    bi = top_i.reshape(B, _NUM_CLASSES, _PRESELECT)
    bb = boxes_t[jnp.arange(B)[:, None, None, None],
                 jnp.arange(4)[None, None, :, None],
                 bi[:, :, None, :]]
    s_pad = jnp.zeros((B, _NUM_CLASSES, _PAD), jnp.float32)
    s_pad = s_pad.at[:, :, :_PRESELECT].set(
        top_s.reshape(B, _NUM_CLASSES, _PRESELECT))
    b_pad = jnp.zeros((B, _NUM_CLASSES, 4, _PAD), jnp.float32)
    b_pad = b_pad.at[:, :, :, :_PRESELECT].set(bb)
    x1p = b_pad[:, :, 0, :]
    y1p = b_pad[:, :, 1, :]
    x2p = b_pad[:, :, 2, :]
    y2p = b_pad[:, :, 3, :]

    spec_in = pl.BlockSpec((1, _NUM_CLASSES, _PAD), lambda b: (b, 0, 0))
    spec_out = pl.BlockSpec((1, _NUM_CLASSES, 128), lambda b: (b, 0, 0))
    outs = pl.pallas_call(
        _nms_kernel,
        grid=(B,),
        in_specs=[spec_in] * 5,
        out_specs=[spec_out] * 5,
        out_shape=[jax.ShapeDtypeStruct((B, _NUM_CLASSES, 128), jnp.float32)
                   ] * 5,
    )(s_pad, x1p, y1p, x2p, y2p)
    ss, ox1, oy1, ox2, oy2 = outs

    flat_s = ss[:, :, :_MAX_OUT].reshape(B, _NUM_CLASSES * _MAX_OUT)
    flat_b = jnp.stack([ox1, oy1, ox2, oy2], axis=-1)[:, :, :_MAX_OUT, :]
    flat_b = flat_b.reshape(B, _NUM_CLASSES * _MAX_OUT, 4)
    top_s2, top_i2 = jax.lax.top_k(flat_s, _MAX_OUT)
    top_b = jnp.take_along_axis(flat_b, top_i2[:, :, None], axis=1)
    top_c = (top_i2 // _MAX_OUT).astype(jnp.float32)
    ok = top_s2 > 0.0
    top_b = jnp.where(ok[:, :, None], top_b, 0.0)
    top_c = jnp.where(ok, top_c, 0.0)
    top_s2 = jnp.where(ok, top_s2, 0.0)
    valid = jnp.sum(ok, axis=1).astype(jnp.int32)
    return top_b, top_s2, top_c, valid


# speedup vs baseline: 7.4384x; 1.1525x over previous
import functools

import jax
import jax.numpy as jnp
import numpy as np
from jax.experimental import pallas as pl

_NUM_CLASSES = 80
_ANCHORS = np.array(
    [[10, 13], [16, 30], [33, 23], [30, 61], [62, 45], [59, 119],
     [116, 90], [156, 198], [373, 326]], dtype=np.float32)
_MASK = [[0, 1, 2], [3, 4, 5], [6, 7, 8]]
_STRIDES = [8.0, 16.0, 32.0]
_MAX_OUT = 100
_IOU_THR = 0.5
_SCORE_THR = 0.25
_PRESELECT = 400
_PAD = 512

_LEVELS = [(64, 64), (32, 32), (16, 16)]
_N_TOTAL = sum(h * w * 3 for h, w in _LEVELS)


def _build_consts():
    gx, gy, aw, ah, st = [], [], [], [], []
    for i, (H, W) in enumerate(_LEVELS):
        anc = _ANCHORS[_MASK[i]]
        hh, ww, aa = np.meshgrid(np.arange(H), np.arange(W), np.arange(3),
                                 indexing="ij")
        gx.append(ww.reshape(-1).astype(np.float32))
        gy.append(hh.reshape(-1).astype(np.float32))
        aw.append(anc[aa.reshape(-1), 0])
        ah.append(anc[aa.reshape(-1), 1])
        st.append(np.full(H * W * 3, _STRIDES[i], dtype=np.float32))
    rows = [np.concatenate(v) for v in (gx, gy, aw, ah, st)]
    rows += [np.zeros(_N_TOTAL, np.float32)] * 3
    return np.stack(rows, axis=0)


_CONSTS = _build_consts()


_BITS_LO = 0x3E800000
_BITS_HI = 0x3F800000


def _decode_kernel(x_ref, c_ref, b_ref, s_ref, t_ref):
    x = x_ref[0]
    c = c_ref[...]
    xy = jax.nn.sigmoid(x[0:2, :])
    ctr = (xy + c[0:2, :]) * c[4:5, :]
    half = jnp.exp(x[2:4, :]) * c[2:4, :] * 0.5
    b_ref[0, 0:2, :] = ctr - half
    b_ref[0, 2:4, :] = ctr + half
    obj = jax.nn.sigmoid(x[4:5, :])
    sc = obj * jax.nn.sigmoid(x[5:85, :])
    s_ref[0] = sc

    ms = jnp.where(sc > _SCORE_THR, sc, -1.0)
    lo0 = jnp.full((_NUM_CLASSES, 1), _BITS_LO, jnp.int32)
    hi0 = jnp.full((_NUM_CLASSES, 1), _BITS_HI, jnp.int32)

    def bbody(_, lh):
        lo, hi = lh
        mid = (lo + hi) >> 1
        mid_f = jax.lax.bitcast_convert_type(mid, jnp.float32)
        cnt = jnp.sum((ms >= mid_f).astype(jnp.int32), axis=1, keepdims=True)
        ge = cnt >= _PRESELECT
        return jnp.where(ge, mid, lo), jnp.where(ge, hi, mid)

    lo, _ = jax.lax.fori_loop(0, 24, bbody, (lo0, hi0))
    t = jax.lax.bitcast_convert_type(lo, jnp.float32)
    t_ref[0] = jnp.broadcast_to(t, (_NUM_CLASSES, 128))


_ROWS = 8 * _NUM_CLASSES


def _nms_kernel(s_ref, x1_ref, y1_ref, x2_ref, y2_ref,
                os_ref, ox1_ref, oy1_ref, ox2_ref, oy2_ref):
    s = s_ref[...]
    cur0 = jnp.where(s > _SCORE_THR, s, -1.0)
    x1 = x1_ref[...]
    y1 = y1_ref[...]
    x2 = x2_ref[...]
    y2 = y2_ref[...]
    a2 = (x2 - x1) * (y2 - y1)
    iota = jax.lax.broadcasted_iota(jnp.int32, (_ROWS, _PAD), 1)
    io_m = jax.lax.broadcasted_iota(jnp.int32, (_ROWS, 128), 1)
    zm = jnp.zeros((_ROWS, 128), jnp.float32)

    def body(i, carry):
        cur, os_, o1, o2, o3, o4 = carry
        m = jnp.max(cur, axis=1, keepdims=True)
        sel_first = jnp.min(jnp.where(cur == m, iota, _PAD), axis=1,
                            keepdims=True)
        sel = iota == sel_first
        bx1 = jnp.sum(jnp.where(sel, x1, 0.0), axis=1, keepdims=True)
        by1 = jnp.sum(jnp.where(sel, y1, 0.0), axis=1, keepdims=True)
        bx2 = jnp.sum(jnp.where(sel, x2, 0.0), axis=1, keepdims=True)
        by2 = jnp.sum(jnp.where(sel, y2, 0.0), axis=1, keepdims=True)
        ok = m > 0.0
        wr = io_m == i
        os_ = jnp.where(wr, jnp.where(ok, m, 0.0), os_)
        o1 = jnp.where(wr, jnp.where(ok, bx1, 0.0), o1)
        o2 = jnp.where(wr, jnp.where(ok, by1, 0.0), o2)
        o3 = jnp.where(wr, jnp.where(ok, bx2, 0.0), o3)
        o4 = jnp.where(wr, jnp.where(ok, by2, 0.0), o4)
        iw = jnp.maximum(jnp.minimum(bx2, x2) - jnp.maximum(bx1, x1), 0.0)
        ih = jnp.maximum(jnp.minimum(by2, y2) - jnp.maximum(by1, y1), 0.0)
        inter = iw * ih
        a1 = (bx2 - bx1) * (by2 - by1)
        iou = inter / (a1 + a2 - inter + 1e-9)
        cur = jnp.where(iou > _IOU_THR, -1.0, cur)
        cur = jnp.where(sel, -1.0, cur)
        return (cur, os_, o1, o2, o3, o4)

    cur, os_, o1, o2, o3, o4 = jax.lax.fori_loop(
        0, _MAX_OUT, body, (cur0, zm, zm, zm, zm, zm))
    os_ref[...] = os_
    ox1_ref[...] = o1
    oy1_ref[...] = o2
    ox2_ref[...] = o3
    oy2_ref[...] = o4


@jax.jit
def kernel(logits_0, logits_1, logits_2):
    B = logits_0.shape[0]
    N = _N_TOTAL
    parts = []
    for lg, (H, W) in zip((logits_0, logits_1, logits_2), _LEVELS):
        parts.append(lg.reshape(B, H * W * 3, 85))
    X = jnp.concatenate(parts, axis=1).transpose(0, 2, 1)
    consts = jnp.asarray(_CONSTS)

    boxes_t, scores_t, t_out = pl.pallas_call(
        _decode_kernel,
        grid=(B,),
        in_specs=[
            pl.BlockSpec((1, 85, N), lambda b: (b, 0, 0)),
            pl.BlockSpec((8, N), lambda b: (0, 0)),
        ],
        out_specs=[
            pl.BlockSpec((1, 4, N), lambda b: (b, 0, 0)),
            pl.BlockSpec((1, _NUM_CLASSES, N), lambda b: (b, 0, 0)),
            pl.BlockSpec((1, _NUM_CLASSES, 128), lambda b: (b, 0, 0)),
        ],
        out_shape=[
            jax.ShapeDtypeStruct((B, 4, N), jnp.float32),
            jax.ShapeDtypeStruct((B, _NUM_CLASSES, N), jnp.float32),
            jax.ShapeDtypeStruct((B, _NUM_CLASSES, 128), jnp.float32),
        ],
    )(X, consts)

    R = B * _NUM_CLASSES
    NCH = N // 128
    sflat = scores_t.reshape(R, N)
    t = t_out[:, :, 0].reshape(R, 1)
    mask = (sflat > _SCORE_THR) & (sflat >= t)
    mch = mask.reshape(R, NCH, 128).astype(jnp.bfloat16)
    cnt = jnp.sum(mch, axis=-1, dtype=jnp.float32)
    ccum = jnp.cumsum(cnt, axis=1)
    total = ccum[:, -1:]
    kf = jnp.arange(_PRESELECT, dtype=jnp.float32)
    cmp = (ccum[:, :, None] <= kf[None, None, :]).astype(jnp.bfloat16)
    chunk_idx = jnp.sum(cmp, axis=1, dtype=jnp.float32)
    base = jnp.sum(cnt.astype(jnp.bfloat16)[:, :, None] * cmp, axis=1,
                   dtype=jnp.float32)
    ci = jnp.minimum(chunk_idx, float(NCH - 1))
    onehot = (jnp.arange(NCH, dtype=jnp.float32)[None, None, :]
              == ci[:, :, None]).astype(jnp.bfloat16)
    selc = jnp.einsum('rkc,rcl->rkl', onehot, mch,
                      preferred_element_type=jnp.bfloat16)
    lane_cum = jnp.cumsum(selc, axis=2)
    j = (kf[None, :] - base).astype(jnp.bfloat16)
    lane_idx = jnp.sum((lane_cum <= j[:, :, None]).astype(jnp.bfloat16),
                       axis=2, dtype=jnp.float32)
    validk = kf[None, :] < total
    idx_f = ci * 128.0 + jnp.minimum(lane_idx, 127.0)
    top_i = jnp.where(validk, idx_f, 0.0).astype(jnp.int32)
    top_s = jnp.where(validk, jnp.take_along_axis(sflat, top_i, axis=1), 0.0)
    bi = top_i.reshape(B, _NUM_CLASSES, _PRESELECT)
    bb = boxes_t[jnp.arange(B)[:, None, None, None],
                 jnp.arange(4)[None, None, :, None],
                 bi[:, :, None, :]]
    s_pad = jnp.zeros((B, _NUM_CLASSES, _PAD), jnp.float32)
    s_pad = s_pad.at[:, :, :_PRESELECT].set(
        top_s.reshape(B, _NUM_CLASSES, _PRESELECT))
    b_pad = jnp.zeros((B, _NUM_CLASSES, 4, _PAD), jnp.float32)
    b_pad = b_pad.at[:, :, :, :_PRESELECT].set(bb)
    x1p = b_pad[:, :, 0, :]
    y1p = b_pad[:, :, 1, :]
    x2p = b_pad[:, :, 2, :]
    y2p = b_pad[:, :, 3, :]

    spec_in = pl.BlockSpec((_ROWS, _PAD), lambda: (0, 0))
    spec_out = pl.BlockSpec((_ROWS, 128), lambda: (0, 0))
    flat2 = lambda a: a.reshape(_ROWS, -1)
    outs = pl.pallas_call(
        _nms_kernel,
        in_specs=[spec_in] * 5,
        out_specs=[spec_out] * 5,
        out_shape=[jax.ShapeDtypeStruct((_ROWS, 128), jnp.float32)] * 5,
    )(flat2(s_pad), flat2(x1p), flat2(y1p), flat2(x2p), flat2(y2p))
    ss, ox1, oy1, ox2, oy2 = (o.reshape(B, _NUM_CLASSES, 128) for o in outs)

    flat_s = ss[:, :, :_MAX_OUT].reshape(B, _NUM_CLASSES * _MAX_OUT)
    flat_b = jnp.stack([ox1, oy1, ox2, oy2], axis=-1)[:, :, :_MAX_OUT, :]
    flat_b = flat_b.reshape(B, _NUM_CLASSES * _MAX_OUT, 4)
    top_s2, top_i2 = jax.lax.top_k(flat_s, _MAX_OUT)
    top_b = jnp.take_along_axis(flat_b, top_i2[:, :, None], axis=1)
    top_c = (top_i2 // _MAX_OUT).astype(jnp.float32)
    ok = top_s2 > 0.0
    top_b = jnp.where(ok[:, :, None], top_b, 0.0)
    top_c = jnp.where(ok, top_c, 0.0)
    top_s2 = jnp.where(ok, top_s2, 0.0)
    valid = jnp.sum(ok, axis=1).astype(jnp.int32)
    return top_b, top_s2, top_c, valid


# speedup vs baseline: 7.4413x; 1.0004x over previous
import jax
import jax.numpy as jnp
import numpy as np
from jax.experimental import pallas as pl

_NUM_CLASSES = 80
_ANCHORS = np.array(
    [[10, 13], [16, 30], [33, 23], [30, 61], [62, 45], [59, 119],
     [116, 90], [156, 198], [373, 326]], dtype=np.float32)
_MASK = [[0, 1, 2], [3, 4, 5], [6, 7, 8]]
_STRIDES = [8.0, 16.0, 32.0]
_MAX_OUT = 100
_IOU_THR = 0.5
_SCORE_THR = 0.25
_PRESELECT = 400
_PAD = 512

_LEVELS = [(64, 64), (32, 32), (16, 16)]
_N_TOTAL = sum(h * w * 3 for h, w in _LEVELS)


def _build_consts():
    gx, gy, aw, ah, st = [], [], [], [], []
    for i, (H, W) in enumerate(_LEVELS):
        anc = _ANCHORS[_MASK[i]]
        hh, ww, aa = np.meshgrid(np.arange(H), np.arange(W), np.arange(3),
                                 indexing="ij")
        gx.append(ww.reshape(-1).astype(np.float32))
        gy.append(hh.reshape(-1).astype(np.float32))
        aw.append(anc[aa.reshape(-1), 0])
        ah.append(anc[aa.reshape(-1), 1])
        st.append(np.full(H * W * 3, _STRIDES[i], dtype=np.float32))
    rows = [np.concatenate(v) for v in (gx, gy, aw, ah, st)]
    rows += [np.zeros(_N_TOTAL, np.float32)] * 3
    return np.stack(rows, axis=0)


_CONSTS = _build_consts()


_BITS_LO = 0x3E800000
_BITS_HI = 0x3F800000


def _decode_kernel(x_ref, c_ref, b_ref, s_ref, t_ref):
    x = x_ref[0]
    c = c_ref[...]
    xy = jax.nn.sigmoid(x[0:2, :])
    ctr = (xy + c[0:2, :]) * c[4:5, :]
    half = jnp.exp(x[2:4, :]) * c[2:4, :] * 0.5
    b_ref[0, 0:2, :] = ctr - half
    b_ref[0, 2:4, :] = ctr + half
    obj = jax.nn.sigmoid(x[4:5, :])
    sc = obj * jax.nn.sigmoid(x[5:85, :])
    s_ref[0] = sc

    ms = jnp.where(sc > _SCORE_THR, sc, -1.0)
    lo0 = jnp.full((_NUM_CLASSES, 1), _BITS_LO, jnp.int32)
    hi0 = jnp.full((_NUM_CLASSES, 1), _BITS_HI, jnp.int32)

    def bbody(_, lh):
        lo, hi = lh
        mid = (lo + hi) >> 1
        mid_f = jax.lax.bitcast_convert_type(mid, jnp.float32)
        cnt = jnp.sum((ms >= mid_f).astype(jnp.int32), axis=1, keepdims=True)
        ge = cnt >= _PRESELECT
        return jnp.where(ge, mid, lo), jnp.where(ge, hi, mid)

    lo, _ = jax.lax.fori_loop(0, 24, bbody, (lo0, hi0))
    t = jax.lax.bitcast_convert_type(lo, jnp.float32)
    t_ref[0] = jnp.broadcast_to(t, (_NUM_CLASSES, 128))


_ROWS = 8 * _NUM_CLASSES


def _nms_kernel(s_ref, x1_ref, y1_ref, x2_ref, y2_ref,
                os_ref, ox1_ref, oy1_ref, ox2_ref, oy2_ref):
    s = s_ref[...]
    cur0 = jnp.where(s > _SCORE_THR, s, -1.0)
    x1 = x1_ref[...]
    y1 = y1_ref[...]
    x2 = x2_ref[...]
    y2 = y2_ref[...]
    a2 = (x2 - x1) * (y2 - y1)
    iota = jax.lax.broadcasted_iota(jnp.int32, (_ROWS, _PAD), 1)
    io_m = jax.lax.broadcasted_iota(jnp.int32, (_ROWS, 128), 1)
    zm = jnp.zeros((_ROWS, 128), jnp.float32)

    def body(i, carry):
        cur, os_, o1, o2, o3, o4 = carry
        m = jnp.max(cur, axis=1, keepdims=True)
        sel_first = jnp.min(jnp.where(cur == m, iota, _PAD), axis=1,
                            keepdims=True)
        sel = iota == sel_first
        bx1 = jnp.sum(jnp.where(sel, x1, 0.0), axis=1, keepdims=True)
        by1 = jnp.sum(jnp.where(sel, y1, 0.0), axis=1, keepdims=True)
        bx2 = jnp.sum(jnp.where(sel, x2, 0.0), axis=1, keepdims=True)
        by2 = jnp.sum(jnp.where(sel, y2, 0.0), axis=1, keepdims=True)
        ok = m > 0.0
        wr = io_m == i
        os_ = jnp.where(wr, jnp.where(ok, m, 0.0), os_)
        o1 = jnp.where(wr, jnp.where(ok, bx1, 0.0), o1)
        o2 = jnp.where(wr, jnp.where(ok, by1, 0.0), o2)
        o3 = jnp.where(wr, jnp.where(ok, bx2, 0.0), o3)
        o4 = jnp.where(wr, jnp.where(ok, by2, 0.0), o4)
        iw = jnp.maximum(jnp.minimum(bx2, x2) - jnp.maximum(bx1, x1), 0.0)
        ih = jnp.maximum(jnp.minimum(by2, y2) - jnp.maximum(by1, y1), 0.0)
        inter = iw * ih
        a1 = (bx2 - bx1) * (by2 - by1)
        iou = inter / (a1 + a2 - inter + 1e-9)
        cur = jnp.where(iou > _IOU_THR, -1.0, cur)
        cur = jnp.where(sel, -1.0, cur)
        return (cur, os_, o1, o2, o3, o4)

    cur, os_, o1, o2, o3, o4 = jax.lax.fori_loop(
        0, _MAX_OUT, body, (cur0, zm, zm, zm, zm, zm))
    os_ref[...] = os_
    ox1_ref[...] = o1
    oy1_ref[...] = o2
    ox2_ref[...] = o3
    oy2_ref[...] = o4


@jax.jit
def kernel(logits_0, logits_1, logits_2):
    B = logits_0.shape[0]
    N = _N_TOTAL
    parts = []
    for lg, (H, W) in zip((logits_0, logits_1, logits_2), _LEVELS):
        parts.append(lg.reshape(B, H * W * 3, 85))
    X = jnp.concatenate(parts, axis=1).transpose(0, 2, 1)
    consts = jnp.asarray(_CONSTS)

    boxes_t, scores_t, t_out = pl.pallas_call(
        _decode_kernel,
        grid=(B,),
        in_specs=[
            pl.BlockSpec((1, 85, N), lambda b: (b, 0, 0)),
            pl.BlockSpec((8, N), lambda b: (0, 0)),
        ],
        out_specs=[
            pl.BlockSpec((1, 4, N), lambda b: (b, 0, 0)),
            pl.BlockSpec((1, _NUM_CLASSES, N), lambda b: (b, 0, 0)),
            pl.BlockSpec((1, _NUM_CLASSES, 128), lambda b: (b, 0, 0)),
        ],
        out_shape=[
            jax.ShapeDtypeStruct((B, 4, N), jnp.float32),
            jax.ShapeDtypeStruct((B, _NUM_CLASSES, N), jnp.float32),
            jax.ShapeDtypeStruct((B, _NUM_CLASSES, 128), jnp.float32),
        ],
    )(X, consts)

    R = B * _NUM_CLASSES
    NCH = N // 128
    sflat = scores_t.reshape(R, N)
    t = t_out[:, :, 0].reshape(R, 1)
    mask = (sflat > _SCORE_THR) & (sflat >= t)
    mch = mask.reshape(R, NCH, 128).astype(jnp.bfloat16)
    cnt = jnp.sum(mch, axis=-1, dtype=jnp.float32)
    ccum = jnp.cumsum(cnt, axis=1)
    total = ccum[:, -1:]
    kf = jnp.arange(_PRESELECT, dtype=jnp.float32)
    cmp = (ccum[:, :, None] <= kf[None, None, :]).astype(jnp.bfloat16)
    chunk_idx = jnp.sum(cmp, axis=1, dtype=jnp.float32)
    base = jnp.sum(cnt.astype(jnp.bfloat16)[:, :, None] * cmp, axis=1,
                   dtype=jnp.float32)
    ci = jnp.minimum(chunk_idx, float(NCH - 1))
    onehot = (jnp.arange(NCH, dtype=jnp.float32)[None, None, :]
              == ci[:, :, None]).astype(jnp.bfloat16)
    selc = jnp.einsum('rkc,rcl->rkl', onehot, mch,
                      preferred_element_type=jnp.bfloat16)
    lane_cum = jnp.cumsum(selc, axis=2)
    j = (kf[None, :] - base).astype(jnp.bfloat16)
    lane_idx = jnp.sum((lane_cum <= j[:, :, None]).astype(jnp.bfloat16),
                       axis=2, dtype=jnp.float32)
    validk = kf[None, :] < total
    idx_f = ci * 128.0 + jnp.minimum(lane_idx, 127.0)
    top_i = jnp.where(validk, idx_f, 0.0).astype(jnp.int32)
    top_s = jnp.where(validk, jnp.take_along_axis(sflat, top_i, axis=1), 0.0)
    bi = top_i.reshape(B, _NUM_CLASSES, _PRESELECT)
    bb = boxes_t[jnp.arange(B)[:, None, None, None],
                 jnp.arange(4)[None, None, :, None],
                 bi[:, :, None, :]]
    s_pad = jnp.zeros((B, _NUM_CLASSES, _PAD), jnp.float32)
    s_pad = s_pad.at[:, :, :_PRESELECT].set(
        top_s.reshape(B, _NUM_CLASSES, _PRESELECT))
    b_pad = jnp.zeros((B, _NUM_CLASSES, 4, _PAD), jnp.float32)
    b_pad = b_pad.at[:, :, :, :_PRESELECT].set(bb)
    x1p = b_pad[:, :, 0, :]
    y1p = b_pad[:, :, 1, :]
    x2p = b_pad[:, :, 2, :]
    y2p = b_pad[:, :, 3, :]

    spec_in = pl.BlockSpec((_ROWS, _PAD), lambda: (0, 0))
    spec_out = pl.BlockSpec((_ROWS, 128), lambda: (0, 0))
    flat2 = lambda a: a.reshape(_ROWS, -1)
    outs = pl.pallas_call(
        _nms_kernel,
        in_specs=[spec_in] * 5,
        out_specs=[spec_out] * 5,
        out_shape=[jax.ShapeDtypeStruct((_ROWS, 128), jnp.float32)] * 5,
    )(flat2(s_pad), flat2(x1p), flat2(y1p), flat2(x2p), flat2(y2p))
    ss, ox1, oy1, ox2, oy2 = (o.reshape(B, _NUM_CLASSES, 128) for o in outs)

    flat_s = ss[:, :, :_MAX_OUT].reshape(B, _NUM_CLASSES * _MAX_OUT)
    flat_b = jnp.stack([ox1, oy1, ox2, oy2], axis=-1)[:, :, :_MAX_OUT, :]
    flat_b = flat_b.reshape(B, _NUM_CLASSES * _MAX_OUT, 4)
    top_s2, top_i2 = jax.lax.top_k(flat_s, _MAX_OUT)
    top_b = jnp.take_along_axis(flat_b, top_i2[:, :, None], axis=1)
    top_c = (top_i2 // _MAX_OUT).astype(jnp.float32)
    ok = top_s2 > 0.0
    top_b = jnp.where(ok[:, :, None], top_b, 0.0)
    top_c = jnp.where(ok, top_c, 0.0)
    top_s2 = jnp.where(ok, top_s2, 0.0)
    valid = jnp.sum(ok, axis=1).astype(jnp.int32)
    return top_b, top_s2, top_c, valid
